# baseline (speedup 1.0000x reference)
import functools

import jax
import jax.numpy as jnp
from jax import lax
from jax.experimental import pallas as pl
from jax.experimental.pallas import tpu as pltpu
from jax.experimental.pallas import tpu_sc as plsc

_NUM_NEURONS = 100000
_NUM_EDGES = 6400000
_TARGET_RATE = 0.05

_PAD_NEURONS = 100352
_HALF = _PAD_NEURONS // 2
_TILE_WORDS = _HALF // 16

_NW = 32
_EDGES_PER_W = _NUM_EDGES // _NW
_CHUNK = 10000
_NCHUNK = _EDGES_PER_W // _CHUNK
_NBUF = 4
_VPC = _CHUNK // 16
_HVR = (0, 312, _VPC)
_HED = (0, 4992, _CHUNK)
_UNROLL = 10

_L2C = (-0.07915037, 0.62881573, -2.0810602, 4.02837277, -2.49677377)
_LN2 = 0.6931471805599453


def _scale_bf16_bits(x):
    rate = jnp.maximum(x, 1e-6)
    ratio = _TARGET_RATE / rate
    b = plsc.bitcast(ratio, jnp.int32)
    e = lax.shift_right_arithmetic(b, 23) - 127
    mb = (b & 0x007FFFFF) | 0x3F800000
    m = plsc.bitcast(mb, jnp.float32)
    l2 = _L2C[0]
    for c in _L2C[1:]:
        l2 = l2 * m + c
    ln = _LN2 * (e.astype(jnp.float32) + l2)
    s = jnp.exp(0.1 * ln)
    s = jnp.minimum(jnp.maximum(s, 0.5), 2.0)
    sb = plsc.bitcast(s, jnp.int32)
    rnd = sb + 0x7FFF + (lax.shift_right_logical(sb, 16) & 1)
    return lax.shift_right_logical(rnd, 16)


def _fused_body(ema_hbm, idx_hbm, w_hbm, out_hbm, table_hbm,
                table_v, bufs, sem_t, sems_i, sems_w, sems_o):
    cid = lax.axis_index("c")
    sid = lax.axis_index("s")
    wid = sid * 2 + cid
    base = wid * _EDGES_PER_W
    i_v0, w_v0 = bufs[0]
    _, w_v1 = bufs[1]

    for g in range(2):
        ib, wb = bufs[g + 2]
        pltpu.async_copy(
            idx_hbm.at[pl.ds(base + g * _CHUNK, _CHUNK)], ib, sems_i[g + 2])
        pltpu.async_copy(
            w_hbm.at[pl.ds(base + g * _CHUNK, _CHUNK)], wb, sems_w[g + 2])

    se = sid * _TILE_WORDS
    c_lo = pltpu.async_copy(ema_hbm.at[pl.ds(se, _TILE_WORDS)],
                            w_v0.at[pl.ds(0, _TILE_WORDS)], sems_i[0])
    c_hi = pltpu.async_copy(ema_hbm.at[pl.ds(_HALF + se, _TILE_WORDS)],
                            w_v1.at[pl.ds(0, _TILE_WORDS)], sems_i[1])
    c_lo.wait()
    c_hi.wait()

    @plsc.parallel_loop(0, _TILE_WORDS // 16, unroll=4)
    def _(i):
        sl = pl.ds(i * 16, 16)
        lo = _scale_bf16_bits(w_v0[sl])
        hi = _scale_bf16_bits(w_v1[sl])
        word = lo | lax.shift_left(hi, 16)
        w_v0[sl] = plsc.bitcast(word, jnp.float32)

    tb = cid * _HALF
    pltpu.sync_copy(w_v0.at[pl.ds(0, _TILE_WORDS)],
                    table_hbm.at[pl.ds(tb + se, _TILE_WORDS)])
    plsc.subcore_barrier()

    ct = pltpu.async_copy(table_hbm.at[pl.ds(tb, _HALF)], table_v, sem_t)
    ct.wait()

    def ring_body(g2, carry):
        for u in range(_NBUF):
            g = g2 * _NBUF + u
            b = (u + 2) % _NBUF
            ib, wb = bufs[b]
            pib, pwb = bufs[u]
            off = base + g * _CHUNK
            poff = off + 2 * _CHUNK

            @pl.when(poff < base + _EDGES_PER_W)
            def _():
                @pl.when(g >= 2)
                def _():
                    for h in range(2):
                        pltpu.make_async_copy(
                            pwb.at[pl.ds(_HED[h], _HED[h + 1] - _HED[h])],
                            out_hbm.at[pl.ds(
                                off - 2 * _CHUNK + _HED[h],
                                _HED[h + 1] - _HED[h])],
                            sems_o[u]).wait()
                pltpu.async_copy(idx_hbm.at[pl.ds(poff, _CHUNK)], pib,
                                 sems_i[u])
                pltpu.async_copy(w_hbm.at[pl.ds(poff, _CHUNK)], pwb,
                                 sems_w[u])

            pltpu.make_async_copy(
                idx_hbm.at[pl.ds(off, _CHUNK)], ib, sems_i[b]).wait()
            pltpu.make_async_copy(
                w_hbm.at[pl.ds(off, _CHUNK)], wb, sems_w[b]).wait()

            for h in range(2):
                hbase = _HVR[h]

                @plsc.parallel_loop(hbase, _HVR[h + 1], unroll=_UNROLL)
                def _(i):
                    sl = pl.ds(i * 16, 16)
                    idx = ib[sl]
                    is_hi = idx >= _HALF
                    widx = jnp.where(is_hi, idx - _HALF, idx)
                    g32 = plsc.bitcast(
                        plsc.load_gather(table_v, [widx]), jnp.int32)
                    bits = jnp.where(is_hi, g32 & jnp.int32(-65536),
                                     lax.shift_left(g32, 16))
                    wb[sl] = wb[sl] * plsc.bitcast(bits, jnp.float32)

                pltpu.async_copy(
                    wb.at[pl.ds(_HED[h], _HED[h + 1] - _HED[h])],
                    out_hbm.at[pl.ds(off + _HED[h], _HED[h + 1] - _HED[h])],
                    sems_o[b])
        return carry

    lax.fori_loop(0, _NCHUNK // _NBUF, ring_body, 0)

    for g in range(_NCHUNK - _NBUF, _NCHUNK):
        r = (g + 2) % _NBUF
        for h in range(2):
            pltpu.make_async_copy(
                bufs[r][1].at[pl.ds(_HED[h], _HED[h + 1] - _HED[h])],
                out_hbm.at[pl.ds(
                    base + g * _CHUNK + _HED[h], _HED[h + 1] - _HED[h])],
                sems_o[r]).wait()


@functools.partial(
    pl.kernel,
    out_type=(
        jax.ShapeDtypeStruct((_NUM_EDGES,), jnp.float32),
        jax.ShapeDtypeStruct((2 * _HALF,), jnp.float32),
    ),
    mesh=plsc.VectorSubcoreMesh(core_axis_name="c", subcore_axis_name="s"),
    compiler_params=pltpu.CompilerParams(needs_layout_passes=False),
    scratch_types=[
        pltpu.VMEM((_HALF,), jnp.float32),
        pltpu.VMEM((_CHUNK,), jnp.int32),
        pltpu.VMEM((_CHUNK,), jnp.float32),
        pltpu.VMEM((_CHUNK,), jnp.int32),
        pltpu.VMEM((_CHUNK,), jnp.float32),
        pltpu.VMEM((_CHUNK,), jnp.int32),
        pltpu.VMEM((_CHUNK,), jnp.float32),
        pltpu.VMEM((_CHUNK,), jnp.int32),
        pltpu.VMEM((_CHUNK,), jnp.float32),
        pltpu.SemaphoreType.DMA,
        pltpu.SemaphoreType.DMA,
        pltpu.SemaphoreType.DMA,
        pltpu.SemaphoreType.DMA,
        pltpu.SemaphoreType.DMA,
        pltpu.SemaphoreType.DMA,
        pltpu.SemaphoreType.DMA,
        pltpu.SemaphoreType.DMA,
        pltpu.SemaphoreType.DMA,
        pltpu.SemaphoreType.DMA,
        pltpu.SemaphoreType.DMA,
        pltpu.SemaphoreType.DMA,
        pltpu.SemaphoreType.DMA,
    ],
)
def _sc_fused(ema_hbm, idx_hbm, w_hbm, out_hbm, table_hbm,
              table_v, i0, w0, i1, w1, i2, w2, i3, w3,
              sem_t, si0, si1, si2, si3, sw0, sw1, sw2, sw3,
              so0, so1, so2, so3):
    _fused_body(ema_hbm, idx_hbm, w_hbm, out_hbm, table_hbm,
                table_v, ((i0, w0), (i1, w1), (i2, w2), (i3, w3)),
                sem_t, (si0, si1, si2, si3), (sw0, sw1, sw2, sw3),
                (so0, so1, so2, so3))


def kernel(w_hat, neuron_to_edge_map, firing_rate_ema):
    ema_pad = jnp.pad(firing_rate_ema, (0, _PAD_NEURONS - _NUM_NEURONS))
    out, _ = _sc_fused(ema_pad, neuron_to_edge_map, w_hat)
    return out


# speedup vs baseline: 1.0136x; 1.0136x over previous
import functools

import jax
import jax.numpy as jnp
from jax import lax
from jax.experimental import pallas as pl
from jax.experimental.pallas import tpu as pltpu
from jax.experimental.pallas import tpu_sc as plsc

_NUM_NEURONS = 100000
_NUM_EDGES = 6400000
_TARGET_RATE = 0.05

_PAD_NEURONS = 100352
_HALF = _PAD_NEURONS // 2
_TILE_WORDS = _HALF // 16

_NW = 32
_EDGES_PER_W = _NUM_EDGES // _NW
_CHUNK = 10000
_NCHUNK = _EDGES_PER_W // _CHUNK
_NBUF = 4
_VPC = _CHUNK // 16
_UNROLL = 10

_L2C = (-0.07915037, 0.62881573, -2.0810602, 4.02837277, -2.49677377)
_LN2 = 0.6931471805599453


def _scale_bf16_bits(x):
    rate = jnp.maximum(x, 1e-6)
    ratio = _TARGET_RATE / rate
    b = plsc.bitcast(ratio, jnp.int32)
    e = lax.shift_right_arithmetic(b, 23) - 127
    mb = (b & 0x007FFFFF) | 0x3F800000
    m = plsc.bitcast(mb, jnp.float32)
    l2 = _L2C[0]
    for c in _L2C[1:]:
        l2 = l2 * m + c
    ln = _LN2 * (e.astype(jnp.float32) + l2)
    s = jnp.exp(0.1 * ln)
    s = jnp.minimum(jnp.maximum(s, 0.5), 2.0)
    sb = plsc.bitcast(s, jnp.int32)
    rnd = sb + 0x7FFF + (lax.shift_right_logical(sb, 16) & 1)
    return lax.shift_right_logical(rnd, 16)


def _fused_body(ema_hbm, idx_hbm, w_hbm, out_hbm, table_hbm,
                table_v, bufs, sem_t, sems_i, sems_w, sems_o):
    cid = lax.axis_index("c")
    sid = lax.axis_index("s")
    wid = sid * 2 + cid
    base = wid * _EDGES_PER_W
    i_v0, w_v0 = bufs[0]
    _, w_v1 = bufs[1]

    for g in range(2):
        ib, wb = bufs[g + 2]
        pltpu.async_copy(
            idx_hbm.at[pl.ds(base + g * _CHUNK, _CHUNK)], ib, sems_i[g + 2])
        pltpu.async_copy(
            w_hbm.at[pl.ds(base + g * _CHUNK, _CHUNK)], wb, sems_w[g + 2])

    se = sid * _TILE_WORDS
    c_lo = pltpu.async_copy(ema_hbm.at[pl.ds(se, _TILE_WORDS)],
                            w_v0.at[pl.ds(0, _TILE_WORDS)], sems_i[0])
    c_hi = pltpu.async_copy(ema_hbm.at[pl.ds(_HALF + se, _TILE_WORDS)],
                            w_v1.at[pl.ds(0, _TILE_WORDS)], sems_i[1])
    c_lo.wait()
    c_hi.wait()

    @plsc.parallel_loop(0, _TILE_WORDS // 16, unroll=4)
    def _(i):
        sl = pl.ds(i * 16, 16)
        lo = _scale_bf16_bits(w_v0[sl])
        hi = _scale_bf16_bits(w_v1[sl])
        word = lo | lax.shift_left(hi, 16)
        w_v0[sl] = plsc.bitcast(word, jnp.float32)

    tb = cid * _HALF
    pltpu.sync_copy(w_v0.at[pl.ds(0, _TILE_WORDS)],
                    table_hbm.at[pl.ds(tb + se, _TILE_WORDS)])
    plsc.subcore_barrier()

    ct = pltpu.async_copy(table_hbm.at[pl.ds(tb, _HALF)], table_v, sem_t)
    ct.wait()

    def ring_body(g2, carry):
        for u in range(_NBUF):
            g = g2 * _NBUF + u
            b = (u + 2) % _NBUF
            ib, wb = bufs[b]
            pib, pwb = bufs[u]
            off = base + g * _CHUNK
            poff = off + 2 * _CHUNK

            @pl.when(poff < base + _EDGES_PER_W)
            def _():
                @pl.when(g >= 2)
                def _():
                    pltpu.make_async_copy(
                        pwb, out_hbm.at[pl.ds(off - 2 * _CHUNK, _CHUNK)],
                        sems_o[u]).wait()
                pltpu.async_copy(idx_hbm.at[pl.ds(poff, _CHUNK)], pib,
                                 sems_i[u])
                pltpu.async_copy(w_hbm.at[pl.ds(poff, _CHUNK)], pwb,
                                 sems_w[u])

            pltpu.make_async_copy(
                idx_hbm.at[pl.ds(off, _CHUNK)], ib, sems_i[b]).wait()
            pltpu.make_async_copy(
                w_hbm.at[pl.ds(off, _CHUNK)], wb, sems_w[b]).wait()

            @plsc.parallel_loop(0, _VPC, unroll=_UNROLL)
            def _(i):
                sl = pl.ds(i * 16, 16)
                idx = ib[sl]
                is_hi = idx >= _HALF
                widx = jnp.where(is_hi, idx - _HALF, idx)
                g32 = plsc.bitcast(
                    plsc.load_gather(table_v, [widx]), jnp.int32)
                bits = jnp.where(is_hi, g32 & jnp.int32(-65536),
                                 lax.shift_left(g32, 16))
                wb[sl] = wb[sl] * plsc.bitcast(bits, jnp.float32)

            pltpu.async_copy(wb, out_hbm.at[pl.ds(off, _CHUNK)], sems_o[b])
        return carry

    lax.fori_loop(0, _NCHUNK // _NBUF, ring_body, 0)

    for g in range(_NCHUNK - _NBUF, _NCHUNK):
        r = (g + 2) % _NBUF
        pltpu.make_async_copy(
            bufs[r][1], out_hbm.at[pl.ds(base + g * _CHUNK, _CHUNK)],
            sems_o[r]).wait()


@functools.partial(
    pl.kernel,
    out_type=(
        jax.ShapeDtypeStruct((_NUM_EDGES,), jnp.float32),
        jax.ShapeDtypeStruct((2 * _HALF,), jnp.float32),
    ),
    mesh=plsc.VectorSubcoreMesh(core_axis_name="c", subcore_axis_name="s"),
    compiler_params=pltpu.CompilerParams(needs_layout_passes=False),
    scratch_types=[
        pltpu.VMEM((_HALF,), jnp.float32),
        pltpu.VMEM((_CHUNK,), jnp.int32),
        pltpu.VMEM((_CHUNK,), jnp.float32),
        pltpu.VMEM((_CHUNK,), jnp.int32),
        pltpu.VMEM((_CHUNK,), jnp.float32),
        pltpu.VMEM((_CHUNK,), jnp.int32),
        pltpu.VMEM((_CHUNK,), jnp.float32),
        pltpu.VMEM((_CHUNK,), jnp.int32),
        pltpu.VMEM((_CHUNK,), jnp.float32),
        pltpu.SemaphoreType.DMA,
        pltpu.SemaphoreType.DMA,
        pltpu.SemaphoreType.DMA,
        pltpu.SemaphoreType.DMA,
        pltpu.SemaphoreType.DMA,
        pltpu.SemaphoreType.DMA,
        pltpu.SemaphoreType.DMA,
        pltpu.SemaphoreType.DMA,
        pltpu.SemaphoreType.DMA,
        pltpu.SemaphoreType.DMA,
        pltpu.SemaphoreType.DMA,
        pltpu.SemaphoreType.DMA,
        pltpu.SemaphoreType.DMA,
    ],
)
def _sc_fused(ema_hbm, idx_hbm, w_hbm, out_hbm, table_hbm,
              table_v, i0, w0, i1, w1, i2, w2, i3, w3,
              sem_t, si0, si1, si2, si3, sw0, sw1, sw2, sw3,
              so0, so1, so2, so3):
    _fused_body(ema_hbm, idx_hbm, w_hbm, out_hbm, table_hbm,
                table_v, ((i0, w0), (i1, w1), (i2, w2), (i3, w3)),
                sem_t, (si0, si1, si2, si3), (sw0, sw1, sw2, sw3),
                (so0, so1, so2, so3))


def kernel(w_hat, neuron_to_edge_map, firing_rate_ema):
    ema_pad = jnp.pad(firing_rate_ema, (0, _PAD_NEURONS - _NUM_NEURONS))
    out, _ = _sc_fused(ema_pad, neuron_to_edge_map, w_hat)
    return out
